# baseline (speedup 1.0000x reference)
import jax
import jax.numpy as jnp
from jax.experimental import pallas as pl

D = 128
H = 256
NUM_LAYERS = 2
Z = 0.8
GN_EPS = 1e-5


def _linear(p, x):
    return x @ p[0] + p[1]


def _graphnorm(p, x):
    w, b, ms = p
    mean = jnp.mean(x, axis=0, keepdims=True)
    out = x - mean * ms
    var = jnp.mean(out * out, axis=0, keepdims=True)
    return w * out / jnp.sqrt(var + GN_EPS) + b


def _gat(p, x, edge_index, edge_attr, num_nodes):
    lin_w, att_src, att_dst, lin_edge_w, att_edge, bias = p
    h = x @ lin_w
    src = edge_index[0]
    dst = edge_index[1]
    e = edge_attr @ lin_edge_w
    alpha = (h[src] * att_src).sum(-1) + (h[dst] * att_dst).sum(-1) + (e * att_edge).sum(-1)
    alpha = jax.nn.leaky_relu(alpha, 0.2)
    amax = jax.ops.segment_max(alpha, dst, num_segments=num_nodes)
    amax = jnp.where(jnp.isfinite(amax), amax, 0.0)
    ex = jnp.exp(alpha - amax[dst])
    denom = jax.ops.segment_sum(ex, dst, num_segments=num_nodes)
    coef = ex / (denom[dst] + 1e-16)
    out = jax.ops.segment_sum(h[src] * coef[:, None], dst, num_segments=num_nodes)
    return out + bias


def _glassconv(p, x_, edge_index, edge_attr, mask, num_nodes):
    x1 = jax.nn.relu(_linear(p["trans1"], x_))
    x0 = jax.nn.relu(_linear(p["trans0"], x_))
    m = mask[:, None]
    x = jnp.where(m, Z * x1 + (1.0 - Z) * x0, Z * x0 + (1.0 - Z) * x1)
    x = _gat(p["gat"], x, edge_index, edge_attr, num_nodes)
    x = _graphnorm(p["gn"], x)
    x = jnp.concatenate([x, x_], axis=-1)
    x1 = _linear(p["comb1"], x)
    x0 = _linear(p["comb0"], x)
    return jnp.where(m, Z * x1 + (1.0 - Z) * x0, Z * x0 + (1.0 - Z) * x1)


def _final_linear_kernel(xc_ref, w_ref, b_ref, o_ref):
    o_ref[...] = xc_ref[...] @ w_ref[...] + b_ref[...]


def kernel(x_, edge_index, edge_attr, question_embeddings, subgraph_mask, params):
    n = x_.shape[0]
    m = edge_attr.shape[0]
    x = jax.nn.relu(_linear(params["node_input"], x_))
    e = jax.nn.relu(_linear(params["edge_input"], edge_attr))
    q = jax.nn.relu(_linear(params["question_input"], question_embeddings))[0]
    for layer in range(NUM_LAYERS):
        q_x = jnp.broadcast_to(q, (n, H))
        xc = jax.nn.relu(_linear(params["nq_mix"][layer], jnp.concatenate([x, q_x], axis=1)))
        q_e = jnp.broadcast_to(q, (m, H))
        ec = jax.nn.relu(_linear(params["eq_mix"][layer], jnp.concatenate([e, q_e], axis=1)))
        x = _glassconv(params["convs"][layer], xc, edge_index, ec, subgraph_mask, n)
        x = _graphnorm(params["gns"][layer], x)
    maskf = subgraph_mask.astype(x.dtype)[:, None]
    cnt = jnp.maximum(maskf.sum(), 1.0)
    x_add = jnp.sum(x * maskf, axis=0, keepdims=True)
    x_mean = x_add / cnt
    x_max = jnp.max(jnp.where(subgraph_mask[:, None], x, -jnp.inf), axis=0, keepdims=True)
    xc = jnp.concatenate([x_mean, x_max, x_add], axis=1)
    w, b = params["final"]
    return pl.pallas_call(
        _final_linear_kernel,
        out_shape=jax.ShapeDtypeStruct((1, H), jnp.float32),
    )(xc, w, b[None, :])


# speedup vs baseline: 5.9111x; 5.9111x over previous
import functools

import jax
import jax.numpy as jnp
from jax import lax
from jax.experimental import pallas as pl
from jax.experimental.pallas import tpu as pltpu
from jax.experimental.pallas import tpu_sc as plsc

N = 10000
NPAD = 10240
E = 320000
D = 128
H = 256
Z = 0.8
GN_EPS = 1e-5
NEG = -3.0e38

NC = 2
NS = 16
LANES = 16


EB = 3200


def _edge_alpha_body(ea_ref, we_ref, be_ref, b0_ref, d0_ref, v0_ref,
                     b1_ref, d1_ref, v1_ref, o_ref):
    ea = ea_ref[...]
    e0 = jnp.maximum(ea @ we_ref[...] + be_ref[...], 0.0)
    t0 = jnp.maximum(e0 @ b0_ref[...] + d0_ref[...], 0.0)
    al0 = jnp.sum(t0 * v0_ref[...], axis=1, keepdims=True)
    t1 = jnp.maximum(e0 @ b1_ref[...] + d1_ref[...], 0.0)
    al1 = jnp.sum(t1 * v1_ref[...], axis=1, keepdims=True)
    o_ref[...] = jnp.concatenate([al0, al1], axis=1)


def _edge_alpha(edge_attr, we, be, b0, d0, v0, b1, d1, v1):
    return pl.pallas_call(
        _edge_alpha_body,
        grid=(E // EB,),
        in_specs=[
            pl.BlockSpec((EB, D), lambda i: (i, 0)),
            pl.BlockSpec((D, H), lambda i: (0, 0)),
            pl.BlockSpec((1, H), lambda i: (0, 0)),
            pl.BlockSpec((H, H), lambda i: (0, 0)),
            pl.BlockSpec((1, H), lambda i: (0, 0)),
            pl.BlockSpec((1, H), lambda i: (0, 0)),
            pl.BlockSpec((H, H), lambda i: (0, 0)),
            pl.BlockSpec((1, H), lambda i: (0, 0)),
            pl.BlockSpec((1, H), lambda i: (0, 0)),
        ],
        out_specs=pl.BlockSpec((EB, 2), lambda i: (i, 0)),
        out_shape=jax.ShapeDtypeStruct((E, 2), jnp.float32),
    )(edge_attr, we, be, b0, d0, v0, b1, d1, v1)


NB = 2000


def _node_stage_body(norm, xin_ref, stats_ref, gw_ref, gb_ref, gms_ref,
                     a_ref, c_ref, wt1_ref, bt1_ref, wt0_ref, bt0_ref,
                     lw_ref, asrc_ref, adst_ref, mf_ref,
                     xc_ref, h_ref, as_ref, ad_ref):
    x = xin_ref[...]
    if norm:
        s1 = stats_ref[0:1, :] * (1.0 / N)
        s2 = stats_ref[1:2, :] * (1.0 / N)
        csh = gms_ref[...] * s1
        var = s2 - 2.0 * csh * s1 + csh * csh
        sc = gw_ref[...] * jax.lax.rsqrt(var + GN_EPS)
        sh = gb_ref[...] - sc * csh
        x = x * sc + sh
    xc = jnp.maximum(x @ a_ref[...] + c_ref[...], 0.0)
    xc_ref[...] = xc
    x1 = jnp.maximum(xc @ wt1_ref[...] + bt1_ref[...], 0.0)
    x0 = jnp.maximum(xc @ wt0_ref[...] + bt0_ref[...], 0.0)
    mf = mf_ref[...]
    cm1 = mf * Z + (1.0 - mf) * (1.0 - Z)
    xm = cm1 * x1 + (1.0 - cm1) * x0
    h = xm @ lw_ref[...]
    h_ref[...] = h
    as_ref[...] = jnp.sum(h * asrc_ref[...], axis=1, keepdims=True)
    ad_ref[...] = jnp.sum(h * adst_ref[...], axis=1, keepdims=True)


def _node_stage(xin, stats, gn, a, c, wt1, bt1, wt0, bt0, lw, asrc, adst, mf,
                norm):
    din = xin.shape[1]
    gw, gb, gms = gn
    return pl.pallas_call(
        functools.partial(_node_stage_body, norm),
        grid=(N // NB,),
        in_specs=[
            pl.BlockSpec((NB, din), lambda i: (i, 0)),
            pl.BlockSpec((2, H), lambda i: (0, 0)),
            pl.BlockSpec((1, H), lambda i: (0, 0)),
            pl.BlockSpec((1, H), lambda i: (0, 0)),
            pl.BlockSpec((1, H), lambda i: (0, 0)),
            pl.BlockSpec((din, H), lambda i: (0, 0)),
            pl.BlockSpec((1, H), lambda i: (0, 0)),
            pl.BlockSpec((H, H), lambda i: (0, 0)),
            pl.BlockSpec((1, H), lambda i: (0, 0)),
            pl.BlockSpec((H, H), lambda i: (0, 0)),
            pl.BlockSpec((1, H), lambda i: (0, 0)),
            pl.BlockSpec((H, H), lambda i: (0, 0)),
            pl.BlockSpec((1, H), lambda i: (0, 0)),
            pl.BlockSpec((1, H), lambda i: (0, 0)),
            pl.BlockSpec((NB, 1), lambda i: (i, 0)),
        ],
        out_specs=[
            pl.BlockSpec((NB, H), lambda i: (i, 0)),
            pl.BlockSpec((NB, H), lambda i: (i, 0)),
            pl.BlockSpec((NB, 1), lambda i: (i, 0)),
            pl.BlockSpec((NB, 1), lambda i: (i, 0)),
        ],
        out_shape=[
            jax.ShapeDtypeStruct((N, H), jnp.float32),
            jax.ShapeDtypeStruct((N, H), jnp.float32),
            jax.ShapeDtypeStruct((N, 1), jnp.float32),
            jax.ShapeDtypeStruct((N, 1), jnp.float32),
        ],
    )(xin, stats, gw[None, :], gb[None, :], gms[None, :], a, c[None, :],
      wt1, bt1[None, :], wt0, bt0[None, :], lw, asrc[None, :], adst[None, :],
      mf)


def _gat_stats_body(n0_ref, n1_ref, denp_ref, bias_ref, gat_ref, st_ref):
    den = jnp.sum(denp_ref[...], axis=1, keepdims=True)
    inv = 1.0 / (den + 1e-16)
    gat = jnp.concatenate([n0_ref[...] * inv, n1_ref[...] * inv], axis=1)
    gat = gat + bias_ref[...]
    gat_ref[...] = gat

    @pl.when(pl.program_id(0) == 0)
    def _():
        st_ref[...] = jnp.zeros_like(st_ref)

    st_ref[0:1, :] += jnp.sum(gat, axis=0, keepdims=True)
    st_ref[1:2, :] += jnp.sum(gat * gat, axis=0, keepdims=True)


def _gat_stats(n0, n1, denp_t, bias):
    return pl.pallas_call(
        _gat_stats_body,
        grid=(N // NB,),
        in_specs=[
            pl.BlockSpec((NB, D), lambda i: (i, 0)),
            pl.BlockSpec((NB, D), lambda i: (i, 0)),
            pl.BlockSpec((NB, NS), lambda i: (i, 0)),
            pl.BlockSpec((1, H), lambda i: (0, 0)),
        ],
        out_specs=[
            pl.BlockSpec((NB, H), lambda i: (i, 0)),
            pl.BlockSpec((2, H), lambda i: (0, 0)),
        ],
        out_shape=[
            jax.ShapeDtypeStruct((N, H), jnp.float32),
            jax.ShapeDtypeStruct((2, H), jnp.float32),
        ],
    )(n0, n1, denp_t, bias)


def _comb_body(gat_ref, xc_ref, st_ref, gw_ref, gb_ref, gms_ref,
               w1g_ref, w1x_ref, b1_ref, w0g_ref, w0x_ref, b0_ref, mf_ref,
               xn_ref, st2_ref):
    s1 = st_ref[0:1, :] * (1.0 / N)
    s2 = st_ref[1:2, :] * (1.0 / N)
    csh = gms_ref[...] * s1
    var = s2 - 2.0 * csh * s1 + csh * csh
    sc = gw_ref[...] * jax.lax.rsqrt(var + GN_EPS)
    sh = gb_ref[...] - sc * csh
    xgn = gat_ref[...] * sc + sh
    xc = xc_ref[...]
    c1 = xgn @ w1g_ref[...] + xc @ w1x_ref[...] + b1_ref[...]
    c0 = xgn @ w0g_ref[...] + xc @ w0x_ref[...] + b0_ref[...]
    mf = mf_ref[...]
    cm1 = mf * Z + (1.0 - mf) * (1.0 - Z)
    xn = cm1 * c1 + (1.0 - cm1) * c0
    xn_ref[...] = xn

    @pl.when(pl.program_id(0) == 0)
    def _():
        st2_ref[...] = jnp.zeros_like(st2_ref)

    st2_ref[0:1, :] += jnp.sum(xn, axis=0, keepdims=True)
    st2_ref[1:2, :] += jnp.sum(xn * xn, axis=0, keepdims=True)


def _comb_stage(gat, xc, stats, gn, w1g, w1x, b1, w0g, w0x, b0, mf):
    gw, gb, gms = gn
    return pl.pallas_call(
        _comb_body,
        grid=(N // NB,),
        in_specs=[
            pl.BlockSpec((NB, H), lambda i: (i, 0)),
            pl.BlockSpec((NB, H), lambda i: (i, 0)),
            pl.BlockSpec((2, H), lambda i: (0, 0)),
            pl.BlockSpec((1, H), lambda i: (0, 0)),
            pl.BlockSpec((1, H), lambda i: (0, 0)),
            pl.BlockSpec((1, H), lambda i: (0, 0)),
            pl.BlockSpec((H, H), lambda i: (0, 0)),
            pl.BlockSpec((H, H), lambda i: (0, 0)),
            pl.BlockSpec((1, H), lambda i: (0, 0)),
            pl.BlockSpec((H, H), lambda i: (0, 0)),
            pl.BlockSpec((H, H), lambda i: (0, 0)),
            pl.BlockSpec((1, H), lambda i: (0, 0)),
            pl.BlockSpec((NB, 1), lambda i: (i, 0)),
        ],
        out_specs=[
            pl.BlockSpec((NB, H), lambda i: (i, 0)),
            pl.BlockSpec((2, H), lambda i: (0, 0)),
        ],
        out_shape=[
            jax.ShapeDtypeStruct((N, H), jnp.float32),
            jax.ShapeDtypeStruct((2, H), jnp.float32),
        ],
    )(gat, xc, stats, gw[None, :], gb[None, :], gms[None, :],
      w1g, w1x, b1[None, :], w0g, w0x, b0[None, :], mf)


def _pool_body(xp_ref, st_ref, gw_ref, gb_ref, gms_ref, mf_ref,
               wm_ref, wx_ref, wa_ref, bf_ref, o_ref, acc_ref, cnt_ref):
    s1 = st_ref[0:1, :] * (1.0 / N)
    s2 = st_ref[1:2, :] * (1.0 / N)
    csh = gms_ref[...] * s1
    var = s2 - 2.0 * csh * s1 + csh * csh
    sc = gw_ref[...] * jax.lax.rsqrt(var + GN_EPS)
    sh = gb_ref[...] - sc * csh
    x = xp_ref[...] * sc + sh
    mf = mf_ref[...]

    @pl.when(pl.program_id(0) == 0)
    def _():
        acc_ref[0:1, :] = jnp.zeros((1, H), jnp.float32)
        acc_ref[1:2, :] = jnp.full((1, H), -jnp.inf, jnp.float32)
        cnt_ref[0, 0] = 0.0

    acc_ref[0:1, :] += jnp.sum(x * mf, axis=0, keepdims=True)
    xm = jnp.where(mf > 0.0, x, -jnp.inf)
    acc_ref[1:2, :] = jnp.maximum(acc_ref[1:2, :],
                                  jnp.max(xm, axis=0, keepdims=True))
    cnt_ref[0, 0] += jnp.sum(mf)

    @pl.when(pl.program_id(0) == (N // NB) - 1)
    def _():
        cntc = jnp.maximum(cnt_ref[0, 0], 1.0)
        x_add = acc_ref[0:1, :]
        x_mean = x_add / cntc
        x_max = acc_ref[1:2, :]
        o_ref[...] = (x_mean @ wm_ref[...] + x_max @ wx_ref[...]
                      + x_add @ wa_ref[...] + bf_ref[...])


def _pool_final(xp, stats, gn, mf, wm, wx, wa, bf):
    gw, gb, gms = gn
    return pl.pallas_call(
        _pool_body,
        grid=(N // NB,),
        in_specs=[
            pl.BlockSpec((NB, H), lambda i: (i, 0)),
            pl.BlockSpec((2, H), lambda i: (0, 0)),
            pl.BlockSpec((1, H), lambda i: (0, 0)),
            pl.BlockSpec((1, H), lambda i: (0, 0)),
            pl.BlockSpec((1, H), lambda i: (0, 0)),
            pl.BlockSpec((NB, 1), lambda i: (i, 0)),
            pl.BlockSpec((H, H), lambda i: (0, 0)),
            pl.BlockSpec((H, H), lambda i: (0, 0)),
            pl.BlockSpec((H, H), lambda i: (0, 0)),
            pl.BlockSpec((1, H), lambda i: (0, 0)),
        ],
        out_specs=pl.BlockSpec((1, H), lambda i: (0, 0)),
        out_shape=jax.ShapeDtypeStruct((1, H), jnp.float32),
        scratch_shapes=[
            pltpu.VMEM((2, H), jnp.float32),
            pltpu.SMEM((1, 1), jnp.float32),
        ],
    )(xp, stats, gw[None, :], gb[None, :], gms[None, :], mf, wm, wx, wa,
      bf[None, :])


_MESH = dict(core_axis_name="c", subcore_axis_name="s", num_cores=NC,
             num_subcores=NS)

EW_A = E // (NC * NS)
ITER_A = EW_A // LANES
NSL = NPAD // NS
EW_B = E // NS
CB = 80
NCH_B = EW_B // CB


def _take16(v, idx):
    return lax.gather(
        v, idx[:, None],
        dimension_numbers=lax.GatherDimensionNumbers(
            offset_dims=(), collapsed_slice_dims=(0,), start_index_map=(0,)),
        slice_sizes=(1,), mode=lax.GatherScatterMode.PROMISE_IN_BOUNDS)


def _seg_shift(kk, vv, combine):
    idx = jnp.arange(LANES, dtype=jnp.int32)
    for shv in (1, 2, 4, 8):
        pidx = jnp.maximum(idx - shv, 0)
        kk_s = _take16(kk, pidx)
        vv_s = _take16(vv, pidx)
        vv = jnp.where((kk_s == kk) & (idx >= shv), combine(vv, vv_s), vv)
    nidx = jnp.minimum(idx + 1, LANES - 1)
    kk_n = _take16(kk, nidx)
    islast = (kk_n != kk) | (idx == LANES - 1)
    return vv, islast


def _sc_alpha_amax_body(src_hbm, dst_hbm, eal_hbm, as_hbm, ad_hbm,
                        alpha_hbm, amax2_hbm,
                        sv, dv, ev, asv, adv, abuf, amx, cmb, rbuf, shd):
    ci = lax.axis_index("c")
    si = lax.axis_index("s")
    wid = ci * NS + si
    base = wid * EW_A
    pltpu.sync_copy(src_hbm.at[pl.ds(base, EW_A)], sv)
    pltpu.sync_copy(dst_hbm.at[pl.ds(base, EW_A)], dv)
    pltpu.sync_copy(eal_hbm.at[pl.ds(base, EW_A)], ev)
    pltpu.sync_copy(as_hbm, asv)
    pltpu.sync_copy(ad_hbm, adv)

    def init_body(j, _):
        amx[pl.ds(j * LANES, LANES)] = jnp.full((LANES,), NEG, jnp.float32)
        return 0

    lax.fori_loop(0, NPAD // LANES, init_body, 0)

    def edge_body(i, _):
        sl = pl.ds(i * LANES, LANES)
        s16 = sv[sl]
        d16 = dv[sl]
        e16 = ev[sl]
        a1 = plsc.load_gather(asv, [s16])
        a2 = plsc.load_gather(adv, [d16])
        zv = a1 + a2 + e16
        alpha = jnp.where(zv >= 0.0, zv, 0.2 * zv)
        abuf[sl] = alpha
        kk, vv = plsc.sort_key_val(d16, alpha)
        vmax, islast = _seg_shift(kk, vv, jnp.maximum)
        cur = plsc.load_gather(amx, [kk])
        plsc.store_scatter(amx, [kk], jnp.maximum(cur, vmax), mask=islast)
        return 0

    lax.fori_loop(0, ITER_A, edge_body, 0)

    pltpu.sync_copy(abuf, alpha_hbm.at[pl.ds(base, EW_A)])

    pltpu.sync_copy(amx, shd.at[si])
    plsc.subcore_barrier()
    nb = si * NSL
    for r in range(NS):
        pltpu.sync_copy(shd.at[r, pl.ds(nb, NSL)], cmb.at[r])

    def red_body(j, _):
        sl = pl.ds(j * LANES, LANES)
        acc = cmb[0, sl]
        for r in range(1, NS):
            acc = jnp.maximum(acc, cmb[r, sl])
        rbuf[sl] = acc
        return 0

    lax.fori_loop(0, NSL // LANES, red_body, 0)
    pltpu.sync_copy(rbuf, amax2_hbm.at[ci, pl.ds(nb, NSL)])


def _sc_alpha_amax(src, dst, eal, a_s, a_d):
    mesh = plsc.VectorSubcoreMesh(**_MESH)
    f = pl.kernel(
        _sc_alpha_amax_body,
        out_type=(
            jax.ShapeDtypeStruct((E,), jnp.float32),
            jax.ShapeDtypeStruct((NC, NPAD), jnp.float32),
        ),
        mesh=mesh,
        compiler_params=pltpu.CompilerParams(needs_layout_passes=False),
        scratch_types=[
            pltpu.VMEM((EW_A,), jnp.int32),
            pltpu.VMEM((EW_A,), jnp.int32),
            pltpu.VMEM((EW_A,), jnp.float32),
            pltpu.VMEM((N,), jnp.float32),
            pltpu.VMEM((N,), jnp.float32),
            pltpu.VMEM((EW_A,), jnp.float32),
            pltpu.VMEM((NPAD,), jnp.float32),
            pltpu.VMEM((NS, NSL), jnp.float32),
            pltpu.VMEM((NSL,), jnp.float32),
            pltpu.VMEM_SHARED((NS, NPAD), jnp.float32),
        ],
    )
    return f(src, dst, eal, a_s, a_d)


DEN_CB = 2000
NCH_DEN = EW_B // DEN_CB


def _sc_aggregate_body(src_hbm, dst_hbm, alpha_hbm, amax2_hbm, hst_hbm,
                       num_hbm, denp_hbm,
                       amaxv, denv, rows, idxg, dstc, ac_, exb, dc2, ac2,
                       accum):
    ci = lax.axis_index("c")
    si = lax.axis_index("s")
    base = si * EW_B
    pltpu.sync_copy(amax2_hbm.at[0], rows)

    def ld_body(r, _):
        for v in range(D // LANES):
            slv = pl.ds(v * LANES, LANES)
            amaxv[pl.ds((r * (D // LANES) + v) * LANES, LANES)] = rows[r, slv]
        return 0

    lax.fori_loop(0, CB, ld_body, 0)
    pltpu.sync_copy(amax2_hbm.at[1], rows)

    def mx_body(r, _):
        for v in range(D // LANES):
            slv = pl.ds(v * LANES, LANES)
            sla = pl.ds((r * (D // LANES) + v) * LANES, LANES)
            amaxv[sla] = jnp.maximum(amaxv[sla], rows[r, slv])
        return 0

    lax.fori_loop(0, CB, mx_body, 0)

    def zd_body(j, _):
        denv[pl.ds(j * LANES, LANES)] = jnp.zeros((LANES,), jnp.float32)
        return 0

    lax.fori_loop(0, NPAD // LANES, zd_body, 0)

    def zr_body(r, _):
        for v in range(D // LANES):
            rows[r, pl.ds(v * LANES, LANES)] = jnp.zeros((LANES,), jnp.float32)
        return 0

    lax.fori_loop(0, CB, zr_body, 0)
    nb = si * NSL
    for k in range(NSL // CB):
        pltpu.sync_copy(rows, accum.at[pl.ds(nb + k * CB, CB)])
    plsc.subcore_barrier()

    @pl.when(ci == 0)
    def _():
        def dchunk_body(cc, _):
            eb = base + cc * DEN_CB
            pltpu.sync_copy(dst_hbm.at[pl.ds(eb, DEN_CB)], dc2)
            pltpu.sync_copy(alpha_hbm.at[pl.ds(eb, DEN_CB)], ac2)

            def dg_body(g, _):
                sl16 = pl.ds(g * LANES, LANES)
                d16 = dc2[sl16]
                a16 = ac2[sl16]
                am = plsc.load_gather(amaxv, [d16])
                ex = jnp.exp(a16 - am)
                kk, vv = plsc.sort_key_val(d16, ex)
                vsum, islast = _seg_shift(kk, vv, jnp.add)
                cur = plsc.load_gather(denv, [kk])
                plsc.store_scatter(denv, [kk], cur + vsum, mask=islast)
                return 0

            lax.fori_loop(0, DEN_CB // LANES, dg_body, 0)
            return 0

        lax.fori_loop(0, NCH_DEN, dchunk_body, 0)
        pltpu.sync_copy(denv, denp_hbm.at[si])

    coff = ci * N

    def chunk_body(ch, _):
        eb = base + ch * CB
        pltpu.sync_copy(src_hbm.at[pl.ds(eb, CB)], idxg)
        pltpu.sync_copy(dst_hbm.at[pl.ds(eb, CB)], dstc)
        pltpu.sync_copy(alpha_hbm.at[pl.ds(eb, CB)], ac_)

        def ex_body(j, _):
            sl16 = pl.ds(j * LANES, LANES)
            d16 = dstc[sl16]
            a16 = ac_[sl16]
            idxg[sl16] = idxg[sl16] + coff
            am = plsc.load_gather(amaxv, [d16])
            exb[sl16] = jnp.exp(a16 - am)
            return 0

        lax.fori_loop(0, CB // LANES, ex_body, 0)
        pltpu.sync_copy(hst_hbm.at[idxg], rows)

        def sc_body(r, _):
            esp = plsc.load_gather(exb, [jnp.zeros((LANES,), jnp.int32) + r])
            for v in range(D // LANES):
                slv = pl.ds(v * LANES, LANES)
                rows[r, slv] = rows[r, slv] * esp
            return 0

        lax.fori_loop(0, CB, sc_body, 0)
        pltpu.sync_copy(rows, accum.at[dstc], add=True)
        return 0

    lax.fori_loop(0, NCH_B, chunk_body, 0)
    plsc.subcore_barrier()

    pltpu.sync_copy(accum.at[pl.ds(nb, NSL)], num_hbm.at[ci, pl.ds(nb, NSL)])


def _sc_aggregate(src, dst, alpha, amax2, h_stack):
    mesh = plsc.VectorSubcoreMesh(**_MESH)
    f = pl.kernel(
        _sc_aggregate_body,
        out_type=(
            jax.ShapeDtypeStruct((NC, NPAD, D), jnp.float32),
            jax.ShapeDtypeStruct((NS, NPAD), jnp.float32),
        ),
        mesh=mesh,
        compiler_params=pltpu.CompilerParams(needs_layout_passes=False),
        scratch_types=[
            pltpu.VMEM((NPAD,), jnp.float32),
            pltpu.VMEM((NPAD,), jnp.float32),
            pltpu.VMEM((CB, D), jnp.float32),
            pltpu.VMEM((CB,), jnp.int32),
            pltpu.VMEM((CB,), jnp.int32),
            pltpu.VMEM((CB,), jnp.float32),
            pltpu.VMEM((CB,), jnp.float32),
            pltpu.VMEM((DEN_CB,), jnp.int32),
            pltpu.VMEM((DEN_CB,), jnp.float32),
            pltpu.VMEM_SHARED((NPAD, D), jnp.float32),
        ],
    )
    return f(src, dst, alpha, amax2.reshape(NC, NPAD // D, D), h_stack)


def kernel(x_, edge_index, edge_attr, question_embeddings, subgraph_mask,
           params):
    src = edge_index[0]
    dst = edge_index[1]
    mf = subgraph_mask.astype(jnp.float32)[:, None]

    wq, bq = params["question_input"]
    q = jax.nn.relu(question_embeddings[0, 0] @ wq + bq)

    we, be = params["edge_input"]
    eqw = []
    for l in range(2):
        w, b = params["eq_mix"][l]
        gp = params["convs"][l]["gat"]
        lin_edge_w, att_edge = gp[3], gp[4]
        ve = lin_edge_w @ att_edge
        eqw.append((w[:H], q @ w[H:] + b, ve))

    alphaE = _edge_alpha(edge_attr, we, be[None, :],
                         eqw[0][0], eqw[0][1][None, :], eqw[0][2][None, :],
                         eqw[1][0], eqw[1][1][None, :], eqw[1][2][None, :])

    wn, bn = params["node_input"]
    xin = x_
    stats = jnp.zeros((2, H), jnp.float32)
    gn_prev = (jnp.ones((H,), jnp.float32), jnp.zeros((H,), jnp.float32),
               jnp.ones((H,), jnp.float32))

    out = None
    for l in range(2):
        cv = params["convs"][l]
        wnq, bnq = params["nq_mix"][l]
        lw, a_src, a_dst = cv["gat"][0], cv["gat"][1], cv["gat"][2]
        gat_bias = cv["gat"][5]
        wt0, bt0 = cv["trans0"]
        wt1, bt1 = cv["trans1"]

        if l == 0:
            x0 = _input_relu(x_, wn, bn)
            xin_l = x0
        else:
            xin_l = xin
        a_l = wnq[:H]
        c_l = q @ wnq[H:] + bnq
        xc, h, a_s, a_d = _node_stage(
            xin_l, stats, gn_prev, a_l, c_l, wt1, bt1, wt0, bt0, lw,
            a_src, a_dst, mf, norm=(l == 1))

        h_stack = jnp.concatenate([h[:, :D], h[:, D:]], axis=0)
        alpha, amax2 = _sc_alpha_amax(src, dst, alphaE[:, l], a_s[:, 0],
                                      a_d[:, 0])
        num, denp = _sc_aggregate(src, dst, alpha, amax2, h_stack)

        gat, st_a = _gat_stats(num[0, :N], num[1, :N], denp.T[:N],
                               gat_bias[None, :])
        wc1, bc1 = cv["comb1"]
        wc0, bc0 = cv["comb0"]
        xin, stats = _comb_stage(gat, xc, st_a, cv["gn"],
                                 wc1[:H], wc1[H:], bc1, wc0[:H], wc0[H:], bc0,
                                 mf)
        gn_prev = params["gns"][l]

    wf, bf = params["final"]
    out = _pool_final(xin, stats, gn_prev, mf, wf[:H], wf[H:2 * H], wf[2 * H:],
                      bf)
    return out


def _input_relu_body(x_ref, w_ref, b_ref, o_ref):
    o_ref[...] = jnp.maximum(x_ref[...] @ w_ref[...] + b_ref[...], 0.0)


def _input_relu(x, w, b):
    return pl.pallas_call(
        _input_relu_body,
        grid=(N // NB,),
        in_specs=[
            pl.BlockSpec((NB, D), lambda i: (i, 0)),
            pl.BlockSpec((D, H), lambda i: (0, 0)),
            pl.BlockSpec((1, H), lambda i: (0, 0)),
        ],
        out_specs=pl.BlockSpec((NB, H), lambda i: (i, 0)),
        out_shape=jax.ShapeDtypeStruct((N, H), jnp.float32),
    )(x, w, b[None, :])


# speedup vs baseline: 10.6262x; 1.7977x over previous
import functools

import jax
import jax.numpy as jnp
from jax import lax
from jax.experimental import pallas as pl
from jax.experimental.pallas import tpu as pltpu
from jax.experimental.pallas import tpu_sc as plsc

N = 10000
NPAD = 10240
E = 320000
D = 128
H = 256
Z = 0.8
GN_EPS = 1e-5
NEG = -3.0e38

NC = 2
NS = 16
LANES = 16


EB = 3200


def _edge_alpha_body(ea_ref, we_ref, be_ref, b0_ref, d0_ref, v0_ref,
                     b1_ref, d1_ref, v1_ref, o_ref):
    ea = ea_ref[...]
    e0 = jnp.maximum(ea @ we_ref[...] + be_ref[...], 0.0)
    t0 = jnp.maximum(e0 @ b0_ref[...] + d0_ref[...], 0.0)
    al0 = jnp.sum(t0 * v0_ref[...], axis=1, keepdims=True)
    t1 = jnp.maximum(e0 @ b1_ref[...] + d1_ref[...], 0.0)
    al1 = jnp.sum(t1 * v1_ref[...], axis=1, keepdims=True)
    o_ref[...] = jnp.concatenate([al0, al1], axis=1)


def _edge_alpha(edge_attr, we, be, b0, d0, v0, b1, d1, v1):
    return pl.pallas_call(
        _edge_alpha_body,
        grid=(E // EB,),
        in_specs=[
            pl.BlockSpec((EB, D), lambda i: (i, 0)),
            pl.BlockSpec((D, H), lambda i: (0, 0)),
            pl.BlockSpec((1, H), lambda i: (0, 0)),
            pl.BlockSpec((H, H), lambda i: (0, 0)),
            pl.BlockSpec((1, H), lambda i: (0, 0)),
            pl.BlockSpec((1, H), lambda i: (0, 0)),
            pl.BlockSpec((H, H), lambda i: (0, 0)),
            pl.BlockSpec((1, H), lambda i: (0, 0)),
            pl.BlockSpec((1, H), lambda i: (0, 0)),
        ],
        out_specs=pl.BlockSpec((EB, 2), lambda i: (i, 0)),
        out_shape=jax.ShapeDtypeStruct((E, 2), jnp.float32),
    )(edge_attr, we, be, b0, d0, v0, b1, d1, v1)


NB = 2000


def _node_stage_body(norm, xin_ref, stats_ref, gw_ref, gb_ref, gms_ref,
                     a_ref, c_ref, wt1_ref, bt1_ref, wt0_ref, bt0_ref,
                     lw_ref, asrc_ref, adst_ref, mf_ref,
                     xc_ref, h_ref, as_ref, ad_ref):
    x = xin_ref[...]
    if norm:
        s1 = stats_ref[0:1, :] * (1.0 / N)
        s2 = stats_ref[1:2, :] * (1.0 / N)
        csh = gms_ref[...] * s1
        var = s2 - 2.0 * csh * s1 + csh * csh
        sc = gw_ref[...] * jax.lax.rsqrt(var + GN_EPS)
        sh = gb_ref[...] - sc * csh
        x = x * sc + sh
    xc = jnp.maximum(x @ a_ref[...] + c_ref[...], 0.0)
    xc_ref[...] = xc
    x1 = jnp.maximum(xc @ wt1_ref[...] + bt1_ref[...], 0.0)
    x0 = jnp.maximum(xc @ wt0_ref[...] + bt0_ref[...], 0.0)
    mf = mf_ref[...]
    cm1 = mf * Z + (1.0 - mf) * (1.0 - Z)
    xm = cm1 * x1 + (1.0 - cm1) * x0
    h = xm @ lw_ref[...]
    h_ref[...] = h
    as_ref[...] = jnp.sum(h * asrc_ref[...], axis=1, keepdims=True)
    ad_ref[...] = jnp.sum(h * adst_ref[...], axis=1, keepdims=True)


def _node_stage(xin, stats, gn, a, c, wt1, bt1, wt0, bt0, lw, asrc, adst, mf,
                norm):
    din = xin.shape[1]
    gw, gb, gms = gn
    return pl.pallas_call(
        functools.partial(_node_stage_body, norm),
        grid=(N // NB,),
        in_specs=[
            pl.BlockSpec((NB, din), lambda i: (i, 0)),
            pl.BlockSpec((2, H), lambda i: (0, 0)),
            pl.BlockSpec((1, H), lambda i: (0, 0)),
            pl.BlockSpec((1, H), lambda i: (0, 0)),
            pl.BlockSpec((1, H), lambda i: (0, 0)),
            pl.BlockSpec((din, H), lambda i: (0, 0)),
            pl.BlockSpec((1, H), lambda i: (0, 0)),
            pl.BlockSpec((H, H), lambda i: (0, 0)),
            pl.BlockSpec((1, H), lambda i: (0, 0)),
            pl.BlockSpec((H, H), lambda i: (0, 0)),
            pl.BlockSpec((1, H), lambda i: (0, 0)),
            pl.BlockSpec((H, H), lambda i: (0, 0)),
            pl.BlockSpec((1, H), lambda i: (0, 0)),
            pl.BlockSpec((1, H), lambda i: (0, 0)),
            pl.BlockSpec((NB, 1), lambda i: (i, 0)),
        ],
        out_specs=[
            pl.BlockSpec((NB, H), lambda i: (i, 0)),
            pl.BlockSpec((NB, H), lambda i: (i, 0)),
            pl.BlockSpec((NB, 1), lambda i: (i, 0)),
            pl.BlockSpec((NB, 1), lambda i: (i, 0)),
        ],
        out_shape=[
            jax.ShapeDtypeStruct((N, H), jnp.float32),
            jax.ShapeDtypeStruct((N, H), jnp.float32),
            jax.ShapeDtypeStruct((N, 1), jnp.float32),
            jax.ShapeDtypeStruct((N, 1), jnp.float32),
        ],
    )(xin, stats, gw[None, :], gb[None, :], gms[None, :], a, c[None, :],
      wt1, bt1[None, :], wt0, bt0[None, :], lw, asrc[None, :], adst[None, :],
      mf)


def _gat_stats_body(n0_ref, n1_ref, denp_ref, bias_ref, gat_ref, st_ref):
    den = jnp.sum(denp_ref[...], axis=1, keepdims=True)
    inv = 1.0 / (den + 1e-16)
    gat = jnp.concatenate([n0_ref[...] * inv, n1_ref[...] * inv], axis=1)
    gat = gat + bias_ref[...]
    gat_ref[...] = gat

    @pl.when(pl.program_id(0) == 0)
    def _():
        st_ref[...] = jnp.zeros_like(st_ref)

    st_ref[0:1, :] += jnp.sum(gat, axis=0, keepdims=True)
    st_ref[1:2, :] += jnp.sum(gat * gat, axis=0, keepdims=True)


def _gat_stats(n0, n1, denp_t, bias):
    return pl.pallas_call(
        _gat_stats_body,
        grid=(N // NB,),
        in_specs=[
            pl.BlockSpec((NB, D), lambda i: (i, 0)),
            pl.BlockSpec((NB, D), lambda i: (i, 0)),
            pl.BlockSpec((NB, NC * NS), lambda i: (i, 0)),
            pl.BlockSpec((1, H), lambda i: (0, 0)),
        ],
        out_specs=[
            pl.BlockSpec((NB, H), lambda i: (i, 0)),
            pl.BlockSpec((2, H), lambda i: (0, 0)),
        ],
        out_shape=[
            jax.ShapeDtypeStruct((N, H), jnp.float32),
            jax.ShapeDtypeStruct((2, H), jnp.float32),
        ],
    )(n0, n1, denp_t, bias)


def _comb_body(gat_ref, xc_ref, st_ref, gw_ref, gb_ref, gms_ref,
               w1g_ref, w1x_ref, b1_ref, w0g_ref, w0x_ref, b0_ref, mf_ref,
               xn_ref, st2_ref):
    s1 = st_ref[0:1, :] * (1.0 / N)
    s2 = st_ref[1:2, :] * (1.0 / N)
    csh = gms_ref[...] * s1
    var = s2 - 2.0 * csh * s1 + csh * csh
    sc = gw_ref[...] * jax.lax.rsqrt(var + GN_EPS)
    sh = gb_ref[...] - sc * csh
    xgn = gat_ref[...] * sc + sh
    xc = xc_ref[...]
    c1 = xgn @ w1g_ref[...] + xc @ w1x_ref[...] + b1_ref[...]
    c0 = xgn @ w0g_ref[...] + xc @ w0x_ref[...] + b0_ref[...]
    mf = mf_ref[...]
    cm1 = mf * Z + (1.0 - mf) * (1.0 - Z)
    xn = cm1 * c1 + (1.0 - cm1) * c0
    xn_ref[...] = xn

    @pl.when(pl.program_id(0) == 0)
    def _():
        st2_ref[...] = jnp.zeros_like(st2_ref)

    st2_ref[0:1, :] += jnp.sum(xn, axis=0, keepdims=True)
    st2_ref[1:2, :] += jnp.sum(xn * xn, axis=0, keepdims=True)


def _comb_stage(gat, xc, stats, gn, w1g, w1x, b1, w0g, w0x, b0, mf):
    gw, gb, gms = gn
    return pl.pallas_call(
        _comb_body,
        grid=(N // NB,),
        in_specs=[
            pl.BlockSpec((NB, H), lambda i: (i, 0)),
            pl.BlockSpec((NB, H), lambda i: (i, 0)),
            pl.BlockSpec((2, H), lambda i: (0, 0)),
            pl.BlockSpec((1, H), lambda i: (0, 0)),
            pl.BlockSpec((1, H), lambda i: (0, 0)),
            pl.BlockSpec((1, H), lambda i: (0, 0)),
            pl.BlockSpec((H, H), lambda i: (0, 0)),
            pl.BlockSpec((H, H), lambda i: (0, 0)),
            pl.BlockSpec((1, H), lambda i: (0, 0)),
            pl.BlockSpec((H, H), lambda i: (0, 0)),
            pl.BlockSpec((H, H), lambda i: (0, 0)),
            pl.BlockSpec((1, H), lambda i: (0, 0)),
            pl.BlockSpec((NB, 1), lambda i: (i, 0)),
        ],
        out_specs=[
            pl.BlockSpec((NB, H), lambda i: (i, 0)),
            pl.BlockSpec((2, H), lambda i: (0, 0)),
        ],
        out_shape=[
            jax.ShapeDtypeStruct((N, H), jnp.float32),
            jax.ShapeDtypeStruct((2, H), jnp.float32),
        ],
    )(gat, xc, stats, gw[None, :], gb[None, :], gms[None, :],
      w1g, w1x, b1[None, :], w0g, w0x, b0[None, :], mf)


def _pool_body(xp_ref, st_ref, gw_ref, gb_ref, gms_ref, mf_ref,
               wm_ref, wx_ref, wa_ref, bf_ref, o_ref, acc_ref, cnt_ref):
    s1 = st_ref[0:1, :] * (1.0 / N)
    s2 = st_ref[1:2, :] * (1.0 / N)
    csh = gms_ref[...] * s1
    var = s2 - 2.0 * csh * s1 + csh * csh
    sc = gw_ref[...] * jax.lax.rsqrt(var + GN_EPS)
    sh = gb_ref[...] - sc * csh
    x = xp_ref[...] * sc + sh
    mf = mf_ref[...]

    @pl.when(pl.program_id(0) == 0)
    def _():
        acc_ref[0:1, :] = jnp.zeros((1, H), jnp.float32)
        acc_ref[1:2, :] = jnp.full((1, H), -jnp.inf, jnp.float32)
        cnt_ref[0, 0] = 0.0

    acc_ref[0:1, :] += jnp.sum(x * mf, axis=0, keepdims=True)
    xm = jnp.where(mf > 0.0, x, -jnp.inf)
    acc_ref[1:2, :] = jnp.maximum(acc_ref[1:2, :],
                                  jnp.max(xm, axis=0, keepdims=True))
    cnt_ref[0, 0] += jnp.sum(mf)

    @pl.when(pl.program_id(0) == (N // NB) - 1)
    def _():
        cntc = jnp.maximum(cnt_ref[0, 0], 1.0)
        x_add = acc_ref[0:1, :]
        x_mean = x_add / cntc
        x_max = acc_ref[1:2, :]
        o_ref[...] = (x_mean @ wm_ref[...] + x_max @ wx_ref[...]
                      + x_add @ wa_ref[...] + bf_ref[...])


def _pool_final(xp, stats, gn, mf, wm, wx, wa, bf):
    gw, gb, gms = gn
    return pl.pallas_call(
        _pool_body,
        grid=(N // NB,),
        in_specs=[
            pl.BlockSpec((NB, H), lambda i: (i, 0)),
            pl.BlockSpec((2, H), lambda i: (0, 0)),
            pl.BlockSpec((1, H), lambda i: (0, 0)),
            pl.BlockSpec((1, H), lambda i: (0, 0)),
            pl.BlockSpec((1, H), lambda i: (0, 0)),
            pl.BlockSpec((NB, 1), lambda i: (i, 0)),
            pl.BlockSpec((H, H), lambda i: (0, 0)),
            pl.BlockSpec((H, H), lambda i: (0, 0)),
            pl.BlockSpec((H, H), lambda i: (0, 0)),
            pl.BlockSpec((1, H), lambda i: (0, 0)),
        ],
        out_specs=pl.BlockSpec((1, H), lambda i: (0, 0)),
        out_shape=jax.ShapeDtypeStruct((1, H), jnp.float32),
        scratch_shapes=[
            pltpu.VMEM((2, H), jnp.float32),
            pltpu.SMEM((1, 1), jnp.float32),
        ],
    )(xp, stats, gw[None, :], gb[None, :], gms[None, :], mf, wm, wx, wa,
      bf[None, :])


_MESH = dict(core_axis_name="c", subcore_axis_name="s", num_cores=NC,
             num_subcores=NS)

EW_A = E // (NC * NS)
ITER_A = EW_A // LANES
NSL = NPAD // NS
EW_B = E // NS
CB = 80
NCH_B = EW_B // CB


def _take16(v, idx):
    return lax.gather(
        v, idx[:, None],
        dimension_numbers=lax.GatherDimensionNumbers(
            offset_dims=(), collapsed_slice_dims=(0,), start_index_map=(0,)),
        slice_sizes=(1,), mode=lax.GatherScatterMode.PROMISE_IN_BOUNDS)


def _seg_shift(kk, vv, combine):
    idx = jnp.arange(LANES, dtype=jnp.int32)
    for shv in (1, 2, 4, 8):
        pidx = jnp.maximum(idx - shv, 0)
        kk_s = _take16(kk, pidx)
        vv_s = _take16(vv, pidx)
        vv = jnp.where((kk_s == kk) & (idx >= shv), combine(vv, vv_s), vv)
    nidx = jnp.minimum(idx + 1, LANES - 1)
    kk_n = _take16(kk, nidx)
    islast = (kk_n != kk) | (idx == LANES - 1)
    return vv, islast


def _sc_alpha_amax_body(src_hbm, dst_hbm, eal_hbm, as_hbm, ad_hbm,
                        alpha_hbm, amax2_hbm,
                        sv, dv, ev, asv, adv, abuf, amx, cmb, rbuf, shd):
    ci = lax.axis_index("c")
    si = lax.axis_index("s")
    wid = ci * NS + si
    base = wid * EW_A
    pltpu.sync_copy(src_hbm.at[pl.ds(base, EW_A)], sv)
    pltpu.sync_copy(dst_hbm.at[pl.ds(base, EW_A)], dv)
    pltpu.sync_copy(eal_hbm.at[pl.ds(base, EW_A)], ev)
    pltpu.sync_copy(as_hbm, asv)
    pltpu.sync_copy(ad_hbm, adv)

    def init_body(j, _):
        amx[pl.ds(j * LANES, LANES)] = jnp.full((LANES,), NEG, jnp.float32)
        return 0

    lax.fori_loop(0, NPAD // LANES, init_body, 0)

    def edge_body(i, _):
        sl = pl.ds(i * LANES, LANES)
        s16 = sv[sl]
        d16 = dv[sl]
        e16 = ev[sl]
        a1 = plsc.load_gather(asv, [s16])
        a2 = plsc.load_gather(adv, [d16])
        zv = a1 + a2 + e16
        alpha = jnp.where(zv >= 0.0, zv, 0.2 * zv)
        abuf[sl] = alpha
        kk, vv = plsc.sort_key_val(d16, alpha)
        vmax, islast = _seg_shift(kk, vv, jnp.maximum)
        cur = plsc.load_gather(amx, [kk])
        plsc.store_scatter(amx, [kk], jnp.maximum(cur, vmax), mask=islast)
        return 0

    lax.fori_loop(0, ITER_A, edge_body, 0)

    pltpu.sync_copy(abuf, alpha_hbm.at[pl.ds(base, EW_A)])

    pltpu.sync_copy(amx, shd.at[si])
    plsc.subcore_barrier()
    nb = si * NSL
    for r in range(NS):
        pltpu.sync_copy(shd.at[r, pl.ds(nb, NSL)], cmb.at[r])

    def red_body(j, _):
        sl = pl.ds(j * LANES, LANES)
        acc = cmb[0, sl]
        for r in range(1, NS):
            acc = jnp.maximum(acc, cmb[r, sl])
        rbuf[sl] = acc
        return 0

    lax.fori_loop(0, NSL // LANES, red_body, 0)
    pltpu.sync_copy(rbuf, amax2_hbm.at[ci, pl.ds(nb, NSL)])


def _sc_alpha_amax(src, dst, eal, a_s, a_d):
    mesh = plsc.VectorSubcoreMesh(**_MESH)
    f = pl.kernel(
        _sc_alpha_amax_body,
        out_type=(
            jax.ShapeDtypeStruct((E,), jnp.float32),
            jax.ShapeDtypeStruct((NC, NPAD), jnp.float32),
        ),
        mesh=mesh,
        compiler_params=pltpu.CompilerParams(needs_layout_passes=False),
        scratch_types=[
            pltpu.VMEM((EW_A,), jnp.int32),
            pltpu.VMEM((EW_A,), jnp.int32),
            pltpu.VMEM((EW_A,), jnp.float32),
            pltpu.VMEM((N,), jnp.float32),
            pltpu.VMEM((N,), jnp.float32),
            pltpu.VMEM((EW_A,), jnp.float32),
            pltpu.VMEM((NPAD,), jnp.float32),
            pltpu.VMEM((NS, NSL), jnp.float32),
            pltpu.VMEM((NSL,), jnp.float32),
            pltpu.VMEM_SHARED((NS, NPAD), jnp.float32),
        ],
    )
    return f(src, dst, eal, a_s, a_d)


MB = 4000
NMB = EW_B // MB
PAIRS = (MB // CB) // 2


def _sc_softmax_body(dst_hbm, alpha_hbm, amax2_hbm, ex_hbm, denp_hbm,
                     dv, av, exbuf, amaxv, denv, stg):
    ci = lax.axis_index("c")
    si = lax.axis_index("s")
    wid = ci * NS + si
    base = wid * EW_A
    pltpu.sync_copy(dst_hbm.at[pl.ds(base, EW_A)], dv)
    pltpu.sync_copy(alpha_hbm.at[pl.ds(base, EW_A)], av)

    pltpu.sync_copy(amax2_hbm.at[0], stg)

    def ld_body(r, _):
        for v in range(D // LANES):
            slv = pl.ds(v * LANES, LANES)
            amaxv[pl.ds((r * (D // LANES) + v) * LANES, LANES)] = stg[r, slv]
        return 0

    lax.fori_loop(0, NPAD // D, ld_body, 0)
    pltpu.sync_copy(amax2_hbm.at[1], stg)

    def mx_body(r, _):
        for v in range(D // LANES):
            slv = pl.ds(v * LANES, LANES)
            sla = pl.ds((r * (D // LANES) + v) * LANES, LANES)
            amaxv[sla] = jnp.maximum(amaxv[sla], stg[r, slv])
        return 0

    lax.fori_loop(0, NPAD // D, mx_body, 0)

    def zd_body(j, _):
        denv[pl.ds(j * LANES, LANES)] = jnp.zeros((LANES,), jnp.float32)
        return 0

    lax.fori_loop(0, NPAD // LANES, zd_body, 0)

    def edge_body(i, _):
        sl = pl.ds(i * LANES, LANES)
        d16 = dv[sl]
        a16 = av[sl]
        am = plsc.load_gather(amaxv, [d16])
        ex = jnp.exp(a16 - am)
        exbuf[sl] = ex
        kk, vv = plsc.sort_key_val(d16, ex)
        vsum, islast = _seg_shift(kk, vv, jnp.add)
        cur = plsc.load_gather(denv, [kk])
        plsc.store_scatter(denv, [kk], cur + vsum, mask=islast)
        return 0

    lax.fori_loop(0, ITER_A, edge_body, 0)
    pltpu.sync_copy(exbuf, ex_hbm.at[pl.ds(base, EW_A)])
    pltpu.sync_copy(denv, denp_hbm.at[wid])


def _sc_softmax(dst, alpha, amax2):
    mesh = plsc.VectorSubcoreMesh(**_MESH)
    f = pl.kernel(
        _sc_softmax_body,
        out_type=(
            jax.ShapeDtypeStruct((E,), jnp.float32),
            jax.ShapeDtypeStruct((NC * NS, NPAD), jnp.float32),
        ),
        mesh=mesh,
        compiler_params=pltpu.CompilerParams(needs_layout_passes=False),
        scratch_types=[
            pltpu.VMEM((EW_A,), jnp.int32),
            pltpu.VMEM((EW_A,), jnp.float32),
            pltpu.VMEM((EW_A,), jnp.float32),
            pltpu.VMEM((NPAD,), jnp.float32),
            pltpu.VMEM((NPAD,), jnp.float32),
            pltpu.VMEM((NPAD // D, D), jnp.float32),
        ],
    )
    return f(dst, alpha, amax2.reshape(NC, NPAD // D, D))


def _sc_aggregate_body(src_hbm, dst_hbm, ex_hbm, hst_hbm, num_hbm,
                       srcm, dstm, exm, rows0, rows1, idxg0, idxg1,
                       dstc0, dstc1, accum, sg0, sg1, ss0, ss1):
    ci = lax.axis_index("c")
    si = lax.axis_index("s")
    base = si * EW_B

    def zr_body(r, _):
        for v in range(D // LANES):
            rows0[r, pl.ds(v * LANES, LANES)] = jnp.zeros((LANES,),
                                                          jnp.float32)
        return 0

    lax.fori_loop(0, CB, zr_body, 0)
    nb = si * NSL
    for k in range(NSL // CB):
        pltpu.sync_copy(rows0, accum.at[pl.ds(nb + k * CB, CB)])
    plsc.subcore_barrier()

    coff = ci * N

    def cp80(dst_ref, src_ref, off):
        for j in range(CB // LANES):
            dst_ref[pl.ds(j * LANES, LANES)] = src_ref[pl.ds(off + j * LANES,
                                                             LANES)]

    def scale(rows, exoff):
        def sc_body(r, _):
            esp = plsc.load_gather(
                exm, [jnp.zeros((LANES,), jnp.int32) + (exoff + r)])
            for v in range(D // LANES):
                slv = pl.ds(v * LANES, LANES)
                rows[r, slv] = rows[r, slv] * esp
            return 0

        lax.fori_loop(0, CB, sc_body, 0)

    def meta_body(m, _):
        mb = base + m * MB
        pltpu.sync_copy(src_hbm.at[pl.ds(mb, MB)], srcm)
        pltpu.sync_copy(dst_hbm.at[pl.ds(mb, MB)], dstm)
        pltpu.sync_copy(ex_hbm.at[pl.ds(mb, MB)], exm)

        def off_body(q, _):
            sl = pl.ds(q * LANES, LANES)
            srcm[sl] = srcm[sl] + coff
            return 0

        lax.fori_loop(0, MB // LANES, off_body, 0)

        cp80(idxg0, srcm, 0)
        pltpu.async_copy(hst_hbm.at[idxg0], rows0, sg0)

        def pair_body(p, _):
            ca = 2 * p
            cb = ca + 1
            @pl.when(p > 0)
            def _():
                pltpu.make_async_copy(rows1, accum.at[dstc1], ss1).wait()

            cp80(idxg1, srcm, cb * CB)
            pltpu.async_copy(hst_hbm.at[idxg1], rows1, sg1)

            pltpu.make_async_copy(hst_hbm.at[idxg0], rows0, sg0).wait()
            scale(rows0, ca * CB)
            cp80(dstc0, dstm, ca * CB)
            pltpu.async_copy(rows0, accum.at[dstc0], ss0, add=True)

            pltpu.make_async_copy(hst_hbm.at[idxg1], rows1, sg1).wait()
            scale(rows1, cb * CB)
            cp80(dstc1, dstm, cb * CB)
            pltpu.async_copy(rows1, accum.at[dstc1], ss1, add=True)

            @pl.when(p < PAIRS - 1)
            def _():
                pltpu.make_async_copy(rows0, accum.at[dstc0], ss0).wait()
                cp80(idxg0, srcm, (ca + 2) * CB)
                pltpu.async_copy(hst_hbm.at[idxg0], rows0, sg0)

            return 0

        lax.fori_loop(0, PAIRS, pair_body, 0)
        pltpu.make_async_copy(rows0, accum.at[dstc0], ss0).wait()
        pltpu.make_async_copy(rows1, accum.at[dstc1], ss1).wait()
        return 0

    lax.fori_loop(0, NMB, meta_body, 0)
    plsc.subcore_barrier()

    pltpu.sync_copy(accum.at[pl.ds(nb, NSL)], num_hbm.at[ci, pl.ds(nb, NSL)])


def _sc_aggregate(src, dst, ex, h_stack):
    mesh = plsc.VectorSubcoreMesh(**_MESH)
    f = pl.kernel(
        _sc_aggregate_body,
        out_type=jax.ShapeDtypeStruct((NC, NPAD, D), jnp.float32),
        mesh=mesh,
        compiler_params=pltpu.CompilerParams(needs_layout_passes=False),
        scratch_types=[
            pltpu.VMEM((MB,), jnp.int32),
            pltpu.VMEM((MB,), jnp.int32),
            pltpu.VMEM((MB,), jnp.float32),
            pltpu.VMEM((CB, D), jnp.float32),
            pltpu.VMEM((CB, D), jnp.float32),
            pltpu.VMEM((CB,), jnp.int32),
            pltpu.VMEM((CB,), jnp.int32),
            pltpu.VMEM((CB,), jnp.int32),
            pltpu.VMEM((CB,), jnp.int32),
            pltpu.VMEM_SHARED((NPAD, D), jnp.float32),
            pltpu.SemaphoreType.DMA,
            pltpu.SemaphoreType.DMA,
            pltpu.SemaphoreType.DMA,
            pltpu.SemaphoreType.DMA,
        ],
    )
    return f(src, dst, ex, h_stack)


def kernel(x_, edge_index, edge_attr, question_embeddings, subgraph_mask,
           params):
    src = edge_index[0]
    dst = edge_index[1]
    mf = subgraph_mask.astype(jnp.float32)[:, None]

    wq, bq = params["question_input"]
    q = jax.nn.relu(question_embeddings[0, 0] @ wq + bq)

    we, be = params["edge_input"]
    eqw = []
    for l in range(2):
        w, b = params["eq_mix"][l]
        gp = params["convs"][l]["gat"]
        lin_edge_w, att_edge = gp[3], gp[4]
        ve = lin_edge_w @ att_edge
        eqw.append((w[:H], q @ w[H:] + b, ve))

    alphaE = _edge_alpha(edge_attr, we, be[None, :],
                         eqw[0][0], eqw[0][1][None, :], eqw[0][2][None, :],
                         eqw[1][0], eqw[1][1][None, :], eqw[1][2][None, :])

    wn, bn = params["node_input"]
    xin = x_
    stats = jnp.zeros((2, H), jnp.float32)
    gn_prev = (jnp.ones((H,), jnp.float32), jnp.zeros((H,), jnp.float32),
               jnp.ones((H,), jnp.float32))

    out = None
    for l in range(2):
        cv = params["convs"][l]
        wnq, bnq = params["nq_mix"][l]
        lw, a_src, a_dst = cv["gat"][0], cv["gat"][1], cv["gat"][2]
        gat_bias = cv["gat"][5]
        wt0, bt0 = cv["trans0"]
        wt1, bt1 = cv["trans1"]

        if l == 0:
            x0 = _input_relu(x_, wn, bn)
            xin_l = x0
        else:
            xin_l = xin
        a_l = wnq[:H]
        c_l = q @ wnq[H:] + bnq
        xc, h, a_s, a_d = _node_stage(
            xin_l, stats, gn_prev, a_l, c_l, wt1, bt1, wt0, bt0, lw,
            a_src, a_dst, mf, norm=(l == 1))

        h_stack = jnp.concatenate([h[:, :D], h[:, D:]], axis=0)
        alpha, amax2 = _sc_alpha_amax(src, dst, alphaE[:, l], a_s[:, 0],
                                      a_d[:, 0])
        ex_e, denp = _sc_softmax(dst, alpha, amax2)
        num = _sc_aggregate(src, dst, ex_e, h_stack)

        gat, st_a = _gat_stats(num[0, :N], num[1, :N], denp.T[:N],
                               gat_bias[None, :])
        wc1, bc1 = cv["comb1"]
        wc0, bc0 = cv["comb0"]
        xin, stats = _comb_stage(gat, xc, st_a, cv["gn"],
                                 wc1[:H], wc1[H:], bc1, wc0[:H], wc0[H:], bc0,
                                 mf)
        gn_prev = params["gns"][l]

    wf, bf = params["final"]
    out = _pool_final(xin, stats, gn_prev, mf, wf[:H], wf[H:2 * H], wf[2 * H:],
                      bf)
    return out


def _input_relu_body(x_ref, w_ref, b_ref, o_ref):
    o_ref[...] = jnp.maximum(x_ref[...] @ w_ref[...] + b_ref[...], 0.0)


def _input_relu(x, w, b):
    return pl.pallas_call(
        _input_relu_body,
        grid=(N // NB,),
        in_specs=[
            pl.BlockSpec((NB, D), lambda i: (i, 0)),
            pl.BlockSpec((D, H), lambda i: (0, 0)),
            pl.BlockSpec((1, H), lambda i: (0, 0)),
        ],
        out_specs=pl.BlockSpec((NB, H), lambda i: (i, 0)),
        out_shape=jax.ShapeDtypeStruct((N, H), jnp.float32),
    )(x, w, b[None, :])


# speedup vs baseline: 10.9641x; 1.0318x over previous
import functools

import jax
import jax.numpy as jnp
from jax import lax
from jax.experimental import pallas as pl
from jax.experimental.pallas import tpu as pltpu
from jax.experimental.pallas import tpu_sc as plsc

N = 10000
NPAD = 10240
E = 320000
D = 128
H = 256
Z = 0.8
GN_EPS = 1e-5
NEG = -3.0e38

NC = 2
NS = 16
LANES = 16


EB = 3200


def _edge_alpha_body(ea_ref, we_ref, be_ref, b0_ref, d0_ref, v0_ref,
                     b1_ref, d1_ref, v1_ref, o_ref):
    ea = ea_ref[...]
    e0 = jnp.maximum(ea @ we_ref[...] + be_ref[...], 0.0)
    t0 = jnp.maximum(e0 @ b0_ref[...] + d0_ref[...], 0.0)
    al0 = jnp.sum(t0 * v0_ref[...], axis=1, keepdims=True)
    t1 = jnp.maximum(e0 @ b1_ref[...] + d1_ref[...], 0.0)
    al1 = jnp.sum(t1 * v1_ref[...], axis=1, keepdims=True)
    o_ref[...] = jnp.concatenate([al0, al1], axis=1)


def _edge_alpha(edge_attr, we, be, b0, d0, v0, b1, d1, v1):
    return pl.pallas_call(
        _edge_alpha_body,
        grid=(E // EB,),
        in_specs=[
            pl.BlockSpec((EB, D), lambda i: (i, 0)),
            pl.BlockSpec((D, H), lambda i: (0, 0)),
            pl.BlockSpec((1, H), lambda i: (0, 0)),
            pl.BlockSpec((H, H), lambda i: (0, 0)),
            pl.BlockSpec((1, H), lambda i: (0, 0)),
            pl.BlockSpec((1, H), lambda i: (0, 0)),
            pl.BlockSpec((H, H), lambda i: (0, 0)),
            pl.BlockSpec((1, H), lambda i: (0, 0)),
            pl.BlockSpec((1, H), lambda i: (0, 0)),
        ],
        out_specs=pl.BlockSpec((EB, 2), lambda i: (i, 0)),
        out_shape=jax.ShapeDtypeStruct((E, 2), jnp.float32),
    )(edge_attr, we, be, b0, d0, v0, b1, d1, v1)


NB = 2000


def _node_stage_body(norm, xin_ref, stats_ref, gw_ref, gb_ref, gms_ref,
                     a_ref, c_ref, wt1_ref, bt1_ref, wt0_ref, bt0_ref,
                     lw_ref, asrc_ref, adst_ref, mf_ref,
                     xc_ref, h_ref, as_ref, ad_ref):
    x = xin_ref[...]
    if norm:
        s1 = stats_ref[0:1, :] * (1.0 / N)
        s2 = stats_ref[1:2, :] * (1.0 / N)
        csh = gms_ref[...] * s1
        var = s2 - 2.0 * csh * s1 + csh * csh
        sc = gw_ref[...] * jax.lax.rsqrt(var + GN_EPS)
        sh = gb_ref[...] - sc * csh
        x = x * sc + sh
    xc = jnp.maximum(x @ a_ref[...] + c_ref[...], 0.0)
    xc_ref[...] = xc
    x1 = jnp.maximum(xc @ wt1_ref[...] + bt1_ref[...], 0.0)
    x0 = jnp.maximum(xc @ wt0_ref[...] + bt0_ref[...], 0.0)
    mf = mf_ref[...]
    cm1 = mf * Z + (1.0 - mf) * (1.0 - Z)
    xm = cm1 * x1 + (1.0 - cm1) * x0
    h = xm @ lw_ref[...]
    h_ref[...] = h
    as_ref[...] = jnp.sum(h * asrc_ref[...], axis=1, keepdims=True)
    ad_ref[...] = jnp.sum(h * adst_ref[...], axis=1, keepdims=True)


def _node_stage(xin, stats, gn, a, c, wt1, bt1, wt0, bt0, lw, asrc, adst, mf,
                norm):
    din = xin.shape[1]
    gw, gb, gms = gn
    return pl.pallas_call(
        functools.partial(_node_stage_body, norm),
        grid=(N // NB,),
        in_specs=[
            pl.BlockSpec((NB, din), lambda i: (i, 0)),
            pl.BlockSpec((2, H), lambda i: (0, 0)),
            pl.BlockSpec((1, H), lambda i: (0, 0)),
            pl.BlockSpec((1, H), lambda i: (0, 0)),
            pl.BlockSpec((1, H), lambda i: (0, 0)),
            pl.BlockSpec((din, H), lambda i: (0, 0)),
            pl.BlockSpec((1, H), lambda i: (0, 0)),
            pl.BlockSpec((H, H), lambda i: (0, 0)),
            pl.BlockSpec((1, H), lambda i: (0, 0)),
            pl.BlockSpec((H, H), lambda i: (0, 0)),
            pl.BlockSpec((1, H), lambda i: (0, 0)),
            pl.BlockSpec((H, H), lambda i: (0, 0)),
            pl.BlockSpec((1, H), lambda i: (0, 0)),
            pl.BlockSpec((1, H), lambda i: (0, 0)),
            pl.BlockSpec((NB, 1), lambda i: (i, 0)),
        ],
        out_specs=[
            pl.BlockSpec((NB, H), lambda i: (i, 0)),
            pl.BlockSpec((NB, H), lambda i: (i, 0)),
            pl.BlockSpec((NB, 1), lambda i: (i, 0)),
            pl.BlockSpec((NB, 1), lambda i: (i, 0)),
        ],
        out_shape=[
            jax.ShapeDtypeStruct((N, H), jnp.float32),
            jax.ShapeDtypeStruct((N, H), jnp.float32),
            jax.ShapeDtypeStruct((N, 1), jnp.float32),
            jax.ShapeDtypeStruct((N, 1), jnp.float32),
        ],
    )(xin, stats, gw[None, :], gb[None, :], gms[None, :], a, c[None, :],
      wt1, bt1[None, :], wt0, bt0[None, :], lw, asrc[None, :], adst[None, :],
      mf)


def _gat_stats_body(n0_ref, n1_ref, denp_ref, bias_ref, gat_ref, st_ref):
    den = jnp.sum(denp_ref[...], axis=1, keepdims=True)
    inv = 1.0 / (den + 1e-16)
    gat = jnp.concatenate([n0_ref[...] * inv, n1_ref[...] * inv], axis=1)
    gat = gat + bias_ref[...]
    gat_ref[...] = gat

    @pl.when(pl.program_id(0) == 0)
    def _():
        st_ref[...] = jnp.zeros_like(st_ref)

    st_ref[0:1, :] += jnp.sum(gat, axis=0, keepdims=True)
    st_ref[1:2, :] += jnp.sum(gat * gat, axis=0, keepdims=True)


def _gat_stats(n0, n1, denp_t, bias):
    return pl.pallas_call(
        _gat_stats_body,
        grid=(N // NB,),
        in_specs=[
            pl.BlockSpec((NB, D), lambda i: (i, 0)),
            pl.BlockSpec((NB, D), lambda i: (i, 0)),
            pl.BlockSpec((NB, NC * NS), lambda i: (i, 0)),
            pl.BlockSpec((1, H), lambda i: (0, 0)),
        ],
        out_specs=[
            pl.BlockSpec((NB, H), lambda i: (i, 0)),
            pl.BlockSpec((2, H), lambda i: (0, 0)),
        ],
        out_shape=[
            jax.ShapeDtypeStruct((N, H), jnp.float32),
            jax.ShapeDtypeStruct((2, H), jnp.float32),
        ],
    )(n0, n1, denp_t, bias)


def _comb_body(gat_ref, xc_ref, st_ref, gw_ref, gb_ref, gms_ref,
               w1g_ref, w1x_ref, b1_ref, w0g_ref, w0x_ref, b0_ref, mf_ref,
               xn_ref, st2_ref):
    s1 = st_ref[0:1, :] * (1.0 / N)
    s2 = st_ref[1:2, :] * (1.0 / N)
    csh = gms_ref[...] * s1
    var = s2 - 2.0 * csh * s1 + csh * csh
    sc = gw_ref[...] * jax.lax.rsqrt(var + GN_EPS)
    sh = gb_ref[...] - sc * csh
    xgn = gat_ref[...] * sc + sh
    xc = xc_ref[...]
    c1 = xgn @ w1g_ref[...] + xc @ w1x_ref[...] + b1_ref[...]
    c0 = xgn @ w0g_ref[...] + xc @ w0x_ref[...] + b0_ref[...]
    mf = mf_ref[...]
    cm1 = mf * Z + (1.0 - mf) * (1.0 - Z)
    xn = cm1 * c1 + (1.0 - cm1) * c0
    xn_ref[...] = xn

    @pl.when(pl.program_id(0) == 0)
    def _():
        st2_ref[...] = jnp.zeros_like(st2_ref)

    st2_ref[0:1, :] += jnp.sum(xn, axis=0, keepdims=True)
    st2_ref[1:2, :] += jnp.sum(xn * xn, axis=0, keepdims=True)


def _comb_stage(gat, xc, stats, gn, w1g, w1x, b1, w0g, w0x, b0, mf):
    gw, gb, gms = gn
    return pl.pallas_call(
        _comb_body,
        grid=(N // NB,),
        in_specs=[
            pl.BlockSpec((NB, H), lambda i: (i, 0)),
            pl.BlockSpec((NB, H), lambda i: (i, 0)),
            pl.BlockSpec((2, H), lambda i: (0, 0)),
            pl.BlockSpec((1, H), lambda i: (0, 0)),
            pl.BlockSpec((1, H), lambda i: (0, 0)),
            pl.BlockSpec((1, H), lambda i: (0, 0)),
            pl.BlockSpec((H, H), lambda i: (0, 0)),
            pl.BlockSpec((H, H), lambda i: (0, 0)),
            pl.BlockSpec((1, H), lambda i: (0, 0)),
            pl.BlockSpec((H, H), lambda i: (0, 0)),
            pl.BlockSpec((H, H), lambda i: (0, 0)),
            pl.BlockSpec((1, H), lambda i: (0, 0)),
            pl.BlockSpec((NB, 1), lambda i: (i, 0)),
        ],
        out_specs=[
            pl.BlockSpec((NB, H), lambda i: (i, 0)),
            pl.BlockSpec((2, H), lambda i: (0, 0)),
        ],
        out_shape=[
            jax.ShapeDtypeStruct((N, H), jnp.float32),
            jax.ShapeDtypeStruct((2, H), jnp.float32),
        ],
    )(gat, xc, stats, gw[None, :], gb[None, :], gms[None, :],
      w1g, w1x, b1[None, :], w0g, w0x, b0[None, :], mf)


def _pool_body(xp_ref, st_ref, gw_ref, gb_ref, gms_ref, mf_ref,
               wm_ref, wx_ref, wa_ref, bf_ref, o_ref, acc_ref, cnt_ref):
    s1 = st_ref[0:1, :] * (1.0 / N)
    s2 = st_ref[1:2, :] * (1.0 / N)
    csh = gms_ref[...] * s1
    var = s2 - 2.0 * csh * s1 + csh * csh
    sc = gw_ref[...] * jax.lax.rsqrt(var + GN_EPS)
    sh = gb_ref[...] - sc * csh
    x = xp_ref[...] * sc + sh
    mf = mf_ref[...]

    @pl.when(pl.program_id(0) == 0)
    def _():
        acc_ref[0:1, :] = jnp.zeros((1, H), jnp.float32)
        acc_ref[1:2, :] = jnp.full((1, H), -jnp.inf, jnp.float32)
        cnt_ref[0, 0] = 0.0

    acc_ref[0:1, :] += jnp.sum(x * mf, axis=0, keepdims=True)
    xm = jnp.where(mf > 0.0, x, -jnp.inf)
    acc_ref[1:2, :] = jnp.maximum(acc_ref[1:2, :],
                                  jnp.max(xm, axis=0, keepdims=True))
    cnt_ref[0, 0] += jnp.sum(mf)

    @pl.when(pl.program_id(0) == (N // NB) - 1)
    def _():
        cntc = jnp.maximum(cnt_ref[0, 0], 1.0)
        x_add = acc_ref[0:1, :]
        x_mean = x_add / cntc
        x_max = acc_ref[1:2, :]
        o_ref[...] = (x_mean @ wm_ref[...] + x_max @ wx_ref[...]
                      + x_add @ wa_ref[...] + bf_ref[...])


def _pool_final(xp, stats, gn, mf, wm, wx, wa, bf):
    gw, gb, gms = gn
    return pl.pallas_call(
        _pool_body,
        grid=(N // NB,),
        in_specs=[
            pl.BlockSpec((NB, H), lambda i: (i, 0)),
            pl.BlockSpec((2, H), lambda i: (0, 0)),
            pl.BlockSpec((1, H), lambda i: (0, 0)),
            pl.BlockSpec((1, H), lambda i: (0, 0)),
            pl.BlockSpec((1, H), lambda i: (0, 0)),
            pl.BlockSpec((NB, 1), lambda i: (i, 0)),
            pl.BlockSpec((H, H), lambda i: (0, 0)),
            pl.BlockSpec((H, H), lambda i: (0, 0)),
            pl.BlockSpec((H, H), lambda i: (0, 0)),
            pl.BlockSpec((1, H), lambda i: (0, 0)),
        ],
        out_specs=pl.BlockSpec((1, H), lambda i: (0, 0)),
        out_shape=jax.ShapeDtypeStruct((1, H), jnp.float32),
        scratch_shapes=[
            pltpu.VMEM((2, H), jnp.float32),
            pltpu.SMEM((1, 1), jnp.float32),
        ],
    )(xp, stats, gw[None, :], gb[None, :], gms[None, :], mf, wm, wx, wa,
      bf[None, :])


_MESH = dict(core_axis_name="c", subcore_axis_name="s", num_cores=NC,
             num_subcores=NS)

EW_A = E // (NC * NS)
ITER_A = EW_A // LANES
NSL = NPAD // NS
EW_B = E // NS
CB = 80
NCH_B = EW_B // CB


def _take16(v, idx):
    return lax.gather(
        v, idx[:, None],
        dimension_numbers=lax.GatherDimensionNumbers(
            offset_dims=(), collapsed_slice_dims=(0,), start_index_map=(0,)),
        slice_sizes=(1,), mode=lax.GatherScatterMode.PROMISE_IN_BOUNDS)


def _seg_shift(kk, vv, combine):
    idx = jnp.arange(LANES, dtype=jnp.int32)
    for shv in (1, 2, 4, 8):
        pidx = jnp.maximum(idx - shv, 0)
        kk_s = _take16(kk, pidx)
        vv_s = _take16(vv, pidx)
        vv = jnp.where((kk_s == kk) & (idx >= shv), combine(vv, vv_s), vv)
    nidx = jnp.minimum(idx + 1, LANES - 1)
    kk_n = _take16(kk, nidx)
    islast = (kk_n != kk) | (idx == LANES - 1)
    return vv, islast


def _sc_alpha_amax_body(src_hbm, dst_hbm, eal_hbm, as_hbm, ad_hbm,
                        alpha_hbm, amax2_hbm,
                        sv, dv, ev, asv, adv, abuf, amx, cmb, rbuf, shd):
    ci = lax.axis_index("c")
    si = lax.axis_index("s")
    wid = ci * NS + si
    base = wid * EW_A
    pltpu.sync_copy(src_hbm.at[pl.ds(base, EW_A)], sv)
    pltpu.sync_copy(dst_hbm.at[pl.ds(base, EW_A)], dv)
    pltpu.sync_copy(eal_hbm.at[pl.ds(base, EW_A)], ev)
    pltpu.sync_copy(as_hbm, asv)
    pltpu.sync_copy(ad_hbm, adv)

    def init_body(j, _):
        amx[pl.ds(j * LANES, LANES)] = jnp.full((LANES,), NEG, jnp.float32)
        return 0

    lax.fori_loop(0, NPAD // LANES, init_body, 0)

    def edge_body(i, _):
        sl = pl.ds(i * LANES, LANES)
        s16 = sv[sl]
        d16 = dv[sl]
        e16 = ev[sl]
        a1 = plsc.load_gather(asv, [s16])
        a2 = plsc.load_gather(adv, [d16])
        zv = a1 + a2 + e16
        alpha = jnp.where(zv >= 0.0, zv, 0.2 * zv)
        abuf[sl] = alpha
        kk, vv = plsc.sort_key_val(d16, alpha)
        vmax, islast = _seg_shift(kk, vv, jnp.maximum)
        cur = plsc.load_gather(amx, [kk])
        plsc.store_scatter(amx, [kk], jnp.maximum(cur, vmax), mask=islast)
        return 0

    lax.fori_loop(0, ITER_A, edge_body, 0)

    pltpu.sync_copy(abuf, alpha_hbm.at[pl.ds(base, EW_A)])

    pltpu.sync_copy(amx, shd.at[si])
    plsc.subcore_barrier()
    nb = si * NSL
    for r in range(NS):
        pltpu.sync_copy(shd.at[r, pl.ds(nb, NSL)], cmb.at[r])

    def red_body(j, _):
        sl = pl.ds(j * LANES, LANES)
        acc = cmb[0, sl]
        for r in range(1, NS):
            acc = jnp.maximum(acc, cmb[r, sl])
        rbuf[sl] = acc
        return 0

    lax.fori_loop(0, NSL // LANES, red_body, 0)
    pltpu.sync_copy(rbuf, amax2_hbm.at[ci, pl.ds(nb, NSL)])


def _sc_alpha_amax(src, dst, eal, a_s, a_d):
    mesh = plsc.VectorSubcoreMesh(**_MESH)
    f = pl.kernel(
        _sc_alpha_amax_body,
        out_type=(
            jax.ShapeDtypeStruct((E,), jnp.float32),
            jax.ShapeDtypeStruct((NC, NPAD), jnp.float32),
        ),
        mesh=mesh,
        compiler_params=pltpu.CompilerParams(needs_layout_passes=False),
        scratch_types=[
            pltpu.VMEM((EW_A,), jnp.int32),
            pltpu.VMEM((EW_A,), jnp.int32),
            pltpu.VMEM((EW_A,), jnp.float32),
            pltpu.VMEM((N,), jnp.float32),
            pltpu.VMEM((N,), jnp.float32),
            pltpu.VMEM((EW_A,), jnp.float32),
            pltpu.VMEM((NPAD,), jnp.float32),
            pltpu.VMEM((NS, NSL), jnp.float32),
            pltpu.VMEM((NSL,), jnp.float32),
            pltpu.VMEM_SHARED((NS, NPAD), jnp.float32),
        ],
    )
    return f(src, dst, eal, a_s, a_d)


MB = 4000
NMB = EW_B // MB
PAIRS = (MB // CB) // 2


def _sc_softmax_body(dst_hbm, alpha_hbm, amax2_hbm, ex_hbm, denp_hbm,
                     dv, av, exbuf, amaxv, denv, stg):
    ci = lax.axis_index("c")
    si = lax.axis_index("s")
    wid = ci * NS + si
    base = wid * EW_A
    pltpu.sync_copy(dst_hbm.at[pl.ds(base, EW_A)], dv)
    pltpu.sync_copy(alpha_hbm.at[pl.ds(base, EW_A)], av)

    pltpu.sync_copy(amax2_hbm.at[0], stg)

    def ld_body(r, _):
        for v in range(D // LANES):
            slv = pl.ds(v * LANES, LANES)
            amaxv[pl.ds((r * (D // LANES) + v) * LANES, LANES)] = stg[r, slv]
        return 0

    lax.fori_loop(0, NPAD // D, ld_body, 0)
    pltpu.sync_copy(amax2_hbm.at[1], stg)

    def mx_body(r, _):
        for v in range(D // LANES):
            slv = pl.ds(v * LANES, LANES)
            sla = pl.ds((r * (D // LANES) + v) * LANES, LANES)
            amaxv[sla] = jnp.maximum(amaxv[sla], stg[r, slv])
        return 0

    lax.fori_loop(0, NPAD // D, mx_body, 0)

    def zd_body(j, _):
        denv[pl.ds(j * LANES, LANES)] = jnp.zeros((LANES,), jnp.float32)
        return 0

    lax.fori_loop(0, NPAD // LANES, zd_body, 0)

    def edge_body(i, _):
        sl = pl.ds(i * LANES, LANES)
        d16 = dv[sl]
        a16 = av[sl]
        am = plsc.load_gather(amaxv, [d16])
        ex = jnp.exp(a16 - am)
        exbuf[sl] = ex
        kk, vv = plsc.sort_key_val(d16, ex)
        vsum, islast = _seg_shift(kk, vv, jnp.add)
        cur = plsc.load_gather(denv, [kk])
        plsc.store_scatter(denv, [kk], cur + vsum, mask=islast)
        return 0

    lax.fori_loop(0, ITER_A, edge_body, 0)
    pltpu.sync_copy(exbuf, ex_hbm.at[pl.ds(base, EW_A)])
    pltpu.sync_copy(denv, denp_hbm.at[wid])


def _sc_softmax(dst, alpha, amax2):
    mesh = plsc.VectorSubcoreMesh(**_MESH)
    f = pl.kernel(
        _sc_softmax_body,
        out_type=(
            jax.ShapeDtypeStruct((E,), jnp.float32),
            jax.ShapeDtypeStruct((NC * NS, NPAD), jnp.float32),
        ),
        mesh=mesh,
        compiler_params=pltpu.CompilerParams(needs_layout_passes=False),
        scratch_types=[
            pltpu.VMEM((EW_A,), jnp.int32),
            pltpu.VMEM((EW_A,), jnp.float32),
            pltpu.VMEM((EW_A,), jnp.float32),
            pltpu.VMEM((NPAD,), jnp.float32),
            pltpu.VMEM((NPAD,), jnp.float32),
            pltpu.VMEM((NPAD // D, D), jnp.float32),
        ],
    )
    return f(dst, alpha, amax2.reshape(NC, NPAD // D, D))


def _sc_aggregate_body(src_hbm, dst_hbm, ex_hbm, hst_hbm, num_hbm,
                       srcm, dstm, exm, rows0, rows1, idxg0, idxg1,
                       dstc0, dstc1, accum, sg0, sg1):
    ci = lax.axis_index("c")
    si = lax.axis_index("s")
    base = si * EW_B

    def zr_body(r, _):
        for v in range(D // LANES):
            rows0[r, pl.ds(v * LANES, LANES)] = jnp.zeros((LANES,),
                                                          jnp.float32)
        return 0

    lax.fori_loop(0, CB, zr_body, 0)
    nb = si * NSL
    for k in range(NSL // CB):
        pltpu.sync_copy(rows0, accum.at[pl.ds(nb + k * CB, CB)])
    plsc.subcore_barrier()

    coff = ci * N

    def cp80(dst_ref, src_ref, off):
        for j in range(CB // LANES):
            dst_ref[pl.ds(j * LANES, LANES)] = src_ref[pl.ds(off + j * LANES,
                                                             LANES)]

    def scale(rows, exoff):
        def sc_body(r, _):
            esp = plsc.load_gather(
                exm, [jnp.zeros((LANES,), jnp.int32) + (exoff + r)])
            for v in range(D // LANES):
                slv = pl.ds(v * LANES, LANES)
                rows[r, slv] = rows[r, slv] * esp
            return 0

        lax.fori_loop(0, CB, sc_body, 0)

    def meta_body(m, _):
        mb = base + m * MB
        pltpu.sync_copy(src_hbm.at[pl.ds(mb, MB)], srcm)
        pltpu.sync_copy(dst_hbm.at[pl.ds(mb, MB)], dstm)
        pltpu.sync_copy(ex_hbm.at[pl.ds(mb, MB)], exm)

        def off_body(q, _):
            sl = pl.ds(q * LANES, LANES)
            srcm[sl] = srcm[sl] + coff
            return 0

        lax.fori_loop(0, MB // LANES, off_body, 0)

        cp80(idxg0, srcm, 0)
        pltpu.async_copy(hst_hbm.at[idxg0], rows0, sg0)

        def pair_body(p, _):
            ca = 2 * p
            cb = ca + 1
            cp80(idxg1, srcm, cb * CB)
            pltpu.async_copy(hst_hbm.at[idxg1], rows1, sg1)

            pltpu.make_async_copy(hst_hbm.at[idxg0], rows0, sg0).wait()
            scale(rows0, ca * CB)
            cp80(dstc0, dstm, ca * CB)
            pltpu.sync_copy(rows0, accum.at[dstc0], add=True)

            @pl.when(p < PAIRS - 1)
            def _():
                cp80(idxg0, srcm, (ca + 2) * CB)
                pltpu.async_copy(hst_hbm.at[idxg0], rows0, sg0)

            pltpu.make_async_copy(hst_hbm.at[idxg1], rows1, sg1).wait()
            scale(rows1, cb * CB)
            cp80(dstc1, dstm, cb * CB)
            pltpu.sync_copy(rows1, accum.at[dstc1], add=True)
            return 0

        lax.fori_loop(0, PAIRS, pair_body, 0)
        return 0

    lax.fori_loop(0, NMB, meta_body, 0)
    plsc.subcore_barrier()

    pltpu.sync_copy(accum.at[pl.ds(nb, NSL)], num_hbm.at[ci, pl.ds(nb, NSL)])


def _sc_aggregate(src, dst, ex, h_stack):
    mesh = plsc.VectorSubcoreMesh(**_MESH)
    f = pl.kernel(
        _sc_aggregate_body,
        out_type=jax.ShapeDtypeStruct((NC, NPAD, D), jnp.float32),
        mesh=mesh,
        compiler_params=pltpu.CompilerParams(needs_layout_passes=False),
        scratch_types=[
            pltpu.VMEM((MB,), jnp.int32),
            pltpu.VMEM((MB,), jnp.int32),
            pltpu.VMEM((MB,), jnp.float32),
            pltpu.VMEM((CB, D), jnp.float32),
            pltpu.VMEM((CB, D), jnp.float32),
            pltpu.VMEM((CB,), jnp.int32),
            pltpu.VMEM((CB,), jnp.int32),
            pltpu.VMEM((CB,), jnp.int32),
            pltpu.VMEM((CB,), jnp.int32),
            pltpu.VMEM_SHARED((NPAD, D), jnp.float32),
            pltpu.SemaphoreType.DMA,
            pltpu.SemaphoreType.DMA,
        ],
    )
    return f(src, dst, ex, h_stack)


def kernel(x_, edge_index, edge_attr, question_embeddings, subgraph_mask,
           params):
    src = edge_index[0]
    dst = edge_index[1]
    mf = subgraph_mask.astype(jnp.float32)[:, None]

    wq, bq = params["question_input"]
    q = jax.nn.relu(question_embeddings[0, 0] @ wq + bq)

    we, be = params["edge_input"]
    eqw = []
    for l in range(2):
        w, b = params["eq_mix"][l]
        gp = params["convs"][l]["gat"]
        lin_edge_w, att_edge = gp[3], gp[4]
        ve = lin_edge_w @ att_edge
        eqw.append((w[:H], q @ w[H:] + b, ve))

    alphaE = _edge_alpha(edge_attr, we, be[None, :],
                         eqw[0][0], eqw[0][1][None, :], eqw[0][2][None, :],
                         eqw[1][0], eqw[1][1][None, :], eqw[1][2][None, :])

    wn, bn = params["node_input"]
    xin = x_
    stats = jnp.zeros((2, H), jnp.float32)
    gn_prev = (jnp.ones((H,), jnp.float32), jnp.zeros((H,), jnp.float32),
               jnp.ones((H,), jnp.float32))

    out = None
    for l in range(2):
        cv = params["convs"][l]
        wnq, bnq = params["nq_mix"][l]
        lw, a_src, a_dst = cv["gat"][0], cv["gat"][1], cv["gat"][2]
        gat_bias = cv["gat"][5]
        wt0, bt0 = cv["trans0"]
        wt1, bt1 = cv["trans1"]

        if l == 0:
            x0 = _input_relu(x_, wn, bn)
            xin_l = x0
        else:
            xin_l = xin
        a_l = wnq[:H]
        c_l = q @ wnq[H:] + bnq
        xc, h, a_s, a_d = _node_stage(
            xin_l, stats, gn_prev, a_l, c_l, wt1, bt1, wt0, bt0, lw,
            a_src, a_dst, mf, norm=(l == 1))

        h_stack = jnp.concatenate([h[:, :D], h[:, D:]], axis=0)
        alpha, amax2 = _sc_alpha_amax(src, dst, alphaE[:, l], a_s[:, 0],
                                      a_d[:, 0])
        ex_e, denp = _sc_softmax(dst, alpha, amax2)
        num = _sc_aggregate(src, dst, ex_e, h_stack)

        gat, st_a = _gat_stats(num[0, :N], num[1, :N], denp.T[:N],
                               gat_bias[None, :])
        wc1, bc1 = cv["comb1"]
        wc0, bc0 = cv["comb0"]
        xin, stats = _comb_stage(gat, xc, st_a, cv["gn"],
                                 wc1[:H], wc1[H:], bc1, wc0[:H], wc0[H:], bc0,
                                 mf)
        gn_prev = params["gns"][l]

    wf, bf = params["final"]
    out = _pool_final(xin, stats, gn_prev, mf, wf[:H], wf[H:2 * H], wf[2 * H:],
                      bf)
    return out


def _input_relu_body(x_ref, w_ref, b_ref, o_ref):
    o_ref[...] = jnp.maximum(x_ref[...] @ w_ref[...] + b_ref[...], 0.0)


def _input_relu(x, w, b):
    return pl.pallas_call(
        _input_relu_body,
        grid=(N // NB,),
        in_specs=[
            pl.BlockSpec((NB, D), lambda i: (i, 0)),
            pl.BlockSpec((D, H), lambda i: (0, 0)),
            pl.BlockSpec((1, H), lambda i: (0, 0)),
        ],
        out_specs=pl.BlockSpec((NB, H), lambda i: (i, 0)),
        out_shape=jax.ShapeDtypeStruct((N, H), jnp.float32),
    )(x, w, b[None, :])
